# baseline (speedup 1.0000x reference)
import functools

import jax
import jax.numpy as jnp
from jax import lax
from jax.experimental import pallas as pl
from jax.experimental.pallas import tpu as pltpu
from jax.experimental.pallas import tpu_sc as plsc

OUT_F = 4096
IN_F = 4096
NNZ = 167772

NC = 2
NS = 16
LANES = 16

CHUNKS = 88
SLICE = CHUNKS * 128
NNZ_PAD = SLICE * NS
PAD = NNZ_PAD - NNZ
NROWS2 = NNZ_PAD // 128
NSTREAM = SLICE // 128

HALF_ROWS = OUT_F // 2
SC_ROWS = HALF_ROWS // NC
PASS_ROWS = [128] * 8
PASS_WORDS = [r * IN_F for r in PASS_ROWS]
TILE_WORDS = max(PASS_WORDS)

ZB_WORDS = 16384

BN = 1024


def _sc_body(half_base, rows_hbm, cols_hbm, w_hbm, out_hbm,
             rowsv, colsv, wv, goffv, offsv, zb, shared, sem):
    c = lax.axis_index("c")
    s = lax.axis_index("s")
    core_base = half_base * IN_F + c * (SC_ROWS * IN_F)
    lane = lax.iota(jnp.int32, LANES)

    pltpu.sync_copy(rows_hbm.at[pl.ds(s * CHUNKS, CHUNKS)], rowsv)
    pltpu.sync_copy(cols_hbm.at[pl.ds(s * CHUNKS, CHUNKS)], colsv)
    pltpu.sync_copy(w_hbm.at[pl.ds(s * SLICE, SLICE)], wv)

    @pl.loop(0, ZB_WORDS // LANES)
    def _(i):
        zb[pl.ds(i * LANES, LANES)] = jnp.zeros((LANES,), jnp.float32)

    @pl.loop(0, CHUNKS)
    def _(i):
        for k in range(128 // LANES):
            sl = pl.ds(k * LANES, LANES)
            goffv[i, sl] = (rowsv[i, sl] << 12) + colsv[i, sl] - core_base

    dump = TILE_WORDS + s * LANES + lane

    pass_base = 0
    for p, pwords in enumerate(PASS_WORDS):
        seg = pwords // NS

        for z in range(seg // ZB_WORDS):
            pltpu.sync_copy(zb, shared.at[pl.ds(s * seg + z * ZB_WORDS,
                                                ZB_WORDS)])

        @pl.loop(0, CHUNKS)
        def _(i):
            for k in range(128 // LANES):
                sl = pl.ds(k * LANES, LANES)
                g = (goffv[i, sl] - pass_base).astype(jnp.uint32)
                ok = g < jnp.uint32(pwords)
                offsv[i, sl] = jnp.where(ok, g.astype(jnp.int32), dump)

        plsc.subcore_barrier()

        @pl.loop(0, NSTREAM)
        def _(j):
            pltpu.async_copy(wv.at[pl.ds(j * 128, 128)],
                             shared.at[offsv.at[j]], sem, add=True)

        @pl.loop(0, NSTREAM)
        def _(j):
            pltpu.make_async_copy(wv.at[pl.ds(0, 128)],
                                  shared.at[offsv.at[0]], sem).wait()

        plsc.subcore_barrier()

        out_base = core_base + pass_base + s * seg
        pltpu.sync_copy(shared.at[pl.ds(s * seg, seg)],
                        out_hbm.at[pl.ds(out_base, seg)])

        pass_base += pwords


def _make_sc_scatter(half_base):
    @functools.partial(
        pl.kernel,
        out_type=jax.ShapeDtypeStruct((HALF_ROWS * IN_F,), jnp.float32),
        mesh=plsc.VectorSubcoreMesh(core_axis_name="c", subcore_axis_name="s"),
        scratch_types=[
            pltpu.VMEM((CHUNKS, 128), jnp.int32),
            pltpu.VMEM((CHUNKS, 128), jnp.int32),
            pltpu.VMEM((SLICE,), jnp.float32),
            pltpu.VMEM((CHUNKS, 128), jnp.int32),
            pltpu.VMEM((CHUNKS, 128), jnp.int32),
            pltpu.VMEM((ZB_WORDS,), jnp.float32),
            pltpu.VMEM_SHARED((TILE_WORDS + NS * LANES,), jnp.float32),
            pltpu.SemaphoreType.DMA,
        ],
        name=f"sc_scatter_{half_base}",
    )
    def _sc_scatter(rows_hbm, cols_hbm, w_hbm, out_hbm,
                    rowsv, colsv, wv, goffv, offsv, zb, shared, sem):
        _sc_body(half_base, rows_hbm, cols_hbm, w_hbm, out_hbm,
                 rowsv, colsv, wv, goffv, offsv, zb, shared, sem)

    return _sc_scatter


_sc_scatter_lo = _make_sc_scatter(0)
_sc_scatter_hi = _make_sc_scatter(HALF_ROWS)


def _matmul_body(x_ref, w_ref, b_ref, o_ref):
    xb = x_ref[...]
    wb = w_ref[...].reshape(BN, IN_F).astype(jnp.bfloat16)
    acc = lax.dot_general(xb, wb, (((1,), (1,)), ((), ())),
                          preferred_element_type=jnp.float32)
    o_ref[...] = acc + b_ref[...]


def _matmul_body_alias(x_ref, w_ref, b_ref, _prev_ref, o_ref):
    _matmul_body(x_ref, w_ref, b_ref, o_ref)


def _matmul_lo(x_bf16, w_flat, bias2d):
    m = x_bf16.shape[0]
    grid = (HALF_ROWS // BN,)
    return pl.pallas_call(
        _matmul_body,
        grid=grid,
        in_specs=[
            pl.BlockSpec((m, IN_F), lambda j: (0, 0)),
            pl.BlockSpec((BN * IN_F,), lambda j: (j,)),
            pl.BlockSpec((1, BN), lambda j: (0, j)),
        ],
        out_specs=pl.BlockSpec((m, BN), lambda j: (0, j)),
        out_shape=jax.ShapeDtypeStruct((m, OUT_F), jnp.float32),
    )(x_bf16, w_flat, bias2d)


def _matmul_hi(x_bf16, w_flat, bias2d, prev_out):
    m = x_bf16.shape[0]
    off = HALF_ROWS // BN
    grid = (HALF_ROWS // BN,)
    return pl.pallas_call(
        _matmul_body_alias,
        grid=grid,
        in_specs=[
            pl.BlockSpec((m, IN_F), lambda j: (0, 0)),
            pl.BlockSpec((BN * IN_F,), lambda j: (j,)),
            pl.BlockSpec((1, BN), lambda j: (0, j + off)),
            pl.BlockSpec((8, 128), lambda j: (0, 0)),
        ],
        out_specs=pl.BlockSpec((m, BN), lambda j: (0, j + off)),
        out_shape=jax.ShapeDtypeStruct((m, OUT_F), jnp.float32),
        input_output_aliases={3: 0},
    )(x_bf16, w_flat, bias2d, prev_out)


def kernel(inputs, indices, weights, bias):
    output_shape = list(inputs.shape)
    output_shape[-1] = OUT_F
    x = inputs.reshape(-1, inputs.shape[-1])
    rows = jnp.concatenate(
        [indices[0], jnp.full((PAD,), OUT_F, jnp.int32)]).reshape(NROWS2, 128)
    cols = jnp.concatenate(
        [indices[1], jnp.zeros((PAD,), jnp.int32)]).reshape(NROWS2, 128)
    wvals = jnp.concatenate([weights, jnp.zeros((PAD,), jnp.float32)])
    w_lo = _sc_scatter_lo(rows, cols, wvals)
    w_hi = _sc_scatter_hi(rows, cols, wvals)
    x_bf = x.astype(jnp.bfloat16)
    bias2d = bias.reshape(1, OUT_F)
    out = _matmul_lo(x_bf, w_lo, bias2d)
    out = _matmul_hi(x_bf, w_hi, bias2d, out)
    return out.reshape(output_shape)


# speedup vs baseline: 1.0362x; 1.0362x over previous
import functools

import jax
import jax.numpy as jnp
from jax import lax
from jax.experimental import pallas as pl
from jax.experimental.pallas import tpu as pltpu
from jax.experimental.pallas import tpu_sc as plsc

OUT_F = 4096
IN_F = 4096
NNZ = 167772

NC = 2
NS = 16
LANES = 16

CHUNKS = 88
SLICE = CHUNKS * 128
NNZ_PAD = SLICE * NS
PAD = NNZ_PAD - NNZ
NROWS2 = NNZ_PAD // 128
NSTREAM = SLICE // 128

HALF_ROWS = OUT_F // 2
SC_ROWS = HALF_ROWS // NC
PASS_ROWS = [128] * 8
PASS_WORDS = [r * IN_F for r in PASS_ROWS]
TILE_WORDS = max(PASS_WORDS)

ZB_WORDS = 16384

BN = 256


def _sc_body(half_base, rows_hbm, cols_hbm, w_hbm, out_hbm,
             rowsv, colsv, wv, goffv, offsv, zb, shared, sem):
    c = lax.axis_index("c")
    s = lax.axis_index("s")
    core_base = half_base * IN_F + c * (SC_ROWS * IN_F)
    lane = lax.iota(jnp.int32, LANES)

    pltpu.sync_copy(rows_hbm.at[pl.ds(s * CHUNKS, CHUNKS)], rowsv)
    pltpu.sync_copy(cols_hbm.at[pl.ds(s * CHUNKS, CHUNKS)], colsv)
    pltpu.sync_copy(w_hbm.at[pl.ds(s * SLICE, SLICE)], wv)

    @pl.loop(0, ZB_WORDS // LANES)
    def _(i):
        zb[pl.ds(i * LANES, LANES)] = jnp.zeros((LANES,), jnp.float32)

    @pl.loop(0, CHUNKS)
    def _(i):
        for k in range(128 // LANES):
            sl = pl.ds(k * LANES, LANES)
            goffv[i, sl] = (rowsv[i, sl] << 12) + colsv[i, sl] - core_base

    dump = TILE_WORDS + s * LANES + lane

    pass_base = 0
    for p, pwords in enumerate(PASS_WORDS):
        seg = pwords // NS

        for z in range(seg // ZB_WORDS):
            pltpu.sync_copy(zb, shared.at[pl.ds(s * seg + z * ZB_WORDS,
                                                ZB_WORDS)])

        @pl.loop(0, CHUNKS)
        def _(i):
            for k in range(128 // LANES):
                sl = pl.ds(k * LANES, LANES)
                g = (goffv[i, sl] - pass_base).astype(jnp.uint32)
                ok = g < jnp.uint32(pwords)
                offsv[i, sl] = jnp.where(ok, g.astype(jnp.int32), dump)

        plsc.subcore_barrier()

        @pl.loop(0, NSTREAM)
        def _(j):
            pltpu.async_copy(wv.at[pl.ds(j * 128, 128)],
                             shared.at[offsv.at[j]], sem, add=True)

        @pl.loop(0, NSTREAM)
        def _(j):
            pltpu.make_async_copy(wv.at[pl.ds(0, 128)],
                                  shared.at[offsv.at[0]], sem).wait()

        plsc.subcore_barrier()

        out_base = core_base + pass_base + s * seg
        pltpu.sync_copy(shared.at[pl.ds(s * seg, seg)],
                        out_hbm.at[pl.ds(out_base, seg)])

        pass_base += pwords


def _make_sc_scatter(half_base):
    @functools.partial(
        pl.kernel,
        out_type=jax.ShapeDtypeStruct((HALF_ROWS * IN_F,), jnp.float32),
        mesh=plsc.VectorSubcoreMesh(core_axis_name="c", subcore_axis_name="s"),
        scratch_types=[
            pltpu.VMEM((CHUNKS, 128), jnp.int32),
            pltpu.VMEM((CHUNKS, 128), jnp.int32),
            pltpu.VMEM((SLICE,), jnp.float32),
            pltpu.VMEM((CHUNKS, 128), jnp.int32),
            pltpu.VMEM((CHUNKS, 128), jnp.int32),
            pltpu.VMEM((ZB_WORDS,), jnp.float32),
            pltpu.VMEM_SHARED((TILE_WORDS + NS * LANES,), jnp.float32),
            pltpu.SemaphoreType.DMA,
        ],
        name=f"sc_scatter_{half_base}",
    )
    def _sc_scatter(rows_hbm, cols_hbm, w_hbm, out_hbm,
                    rowsv, colsv, wv, goffv, offsv, zb, shared, sem):
        _sc_body(half_base, rows_hbm, cols_hbm, w_hbm, out_hbm,
                 rowsv, colsv, wv, goffv, offsv, zb, shared, sem)

    return _sc_scatter


_sc_scatter_lo = _make_sc_scatter(0)
_sc_scatter_hi = _make_sc_scatter(HALF_ROWS)


def _matmul_body(x_ref, w_ref, b_ref, o_ref):
    xb = x_ref[...]
    wb = w_ref[...].reshape(BN, IN_F).astype(jnp.bfloat16)
    acc = lax.dot_general(xb, wb, (((1,), (1,)), ((), ())),
                          preferred_element_type=jnp.float32)
    o_ref[...] = acc + b_ref[...]


def _matmul_body_alias(x_ref, w_ref, b_ref, _prev_ref, o_ref):
    _matmul_body(x_ref, w_ref, b_ref, o_ref)


def _matmul_lo(x_bf16, w_flat, bias2d):
    m = x_bf16.shape[0]
    grid = (HALF_ROWS // BN,)
    return pl.pallas_call(
        _matmul_body,
        grid=grid,
        in_specs=[
            pl.BlockSpec((m, IN_F), lambda j: (0, 0)),
            pl.BlockSpec((BN * IN_F,), lambda j: (j,)),
            pl.BlockSpec((1, BN), lambda j: (0, j)),
        ],
        out_specs=pl.BlockSpec((m, BN), lambda j: (0, j)),
        out_shape=jax.ShapeDtypeStruct((m, OUT_F), jnp.float32),
    )(x_bf16, w_flat, bias2d)


def _matmul_hi(x_bf16, w_flat, bias2d, prev_out):
    m = x_bf16.shape[0]
    off = HALF_ROWS // BN
    grid = (HALF_ROWS // BN,)
    return pl.pallas_call(
        _matmul_body_alias,
        grid=grid,
        in_specs=[
            pl.BlockSpec((m, IN_F), lambda j: (0, 0)),
            pl.BlockSpec((BN * IN_F,), lambda j: (j,)),
            pl.BlockSpec((1, BN), lambda j: (0, j + off)),
            pl.BlockSpec((8, 128), lambda j: (0, 0)),
        ],
        out_specs=pl.BlockSpec((m, BN), lambda j: (0, j + off)),
        out_shape=jax.ShapeDtypeStruct((m, OUT_F), jnp.float32),
        input_output_aliases={3: 0},
    )(x_bf16, w_flat, bias2d, prev_out)


def kernel(inputs, indices, weights, bias):
    output_shape = list(inputs.shape)
    output_shape[-1] = OUT_F
    x = inputs.reshape(-1, inputs.shape[-1])
    rows = jnp.concatenate(
        [indices[0], jnp.full((PAD,), OUT_F, jnp.int32)]).reshape(NROWS2, 128)
    cols = jnp.concatenate(
        [indices[1], jnp.zeros((PAD,), jnp.int32)]).reshape(NROWS2, 128)
    wvals = jnp.concatenate([weights, jnp.zeros((PAD,), jnp.float32)])
    w_lo = _sc_scatter_lo(rows, cols, wvals)
    w_hi = _sc_scatter_hi(rows, cols, wvals)
    x_bf = x.astype(jnp.bfloat16)
    bias2d = bias.reshape(1, OUT_F)
    out = _matmul_lo(x_bf, w_lo, bias2d)
    out = _matmul_hi(x_bf, w_hi, bias2d, out)
    return out.reshape(output_shape)
